# Initial kernel scaffold; baseline (speedup 1.0000x reference)
import jax
import jax.numpy as jnp
from jax.experimental import pallas as pl


def kernel(input, theta, class_w):
    raise NotImplementedError("write your pallas kernel here")


# speedup vs baseline: 2.3691x; 2.3691x over previous
import jax
import jax.numpy as jnp
from jax.experimental import pallas as pl
from jax.experimental.pallas import tpu as pltpu

_EPS = 1e-7
_ATMAX = 7.254340845282884


def _pmanifold_body(cw_ref, inp_ref, th_ref, out_ref, carry_ref):
    b = pl.program_id(0)
    _, _, N = inp_ref.shape
    K, Mdim = th_ref.shape

    ch = inp_ref[0]
    hom = ch[0:1, :]
    y0 = ch[1:2, :]
    y1 = ch[2:3, :]

    nz = (hom != 0.0) | (y0 != 0.0) | (y1 != 0.0)
    lane = jax.lax.broadcasted_iota(jnp.int32, (1, N), 1)
    first_zero = jnp.min(jnp.where(nz, N, lane))
    maskf = (lane < first_zero).astype(jnp.float32)

    w = jnp.where(hom > 0.5, cw_ref[1], cw_ref[0]) * maskf

    th = th_ref[...]
    t0sq = th[:, 0:1] * th[:, 0:1]
    t1sq = th[:, 1:2] * th[:, 1:2]

    y0sq = y0 * y0
    y1sq = y1 * y1

    nu2 = t0sq * y0sq + t1sq * y1sq
    nu = jnp.sqrt(nu2)
    r = jnp.sqrt(1.0 + nu2)
    f = 1.0 / (1.0 + r)
    nx = nu * f
    half_asinh = 0.5 * jnp.log(nu + r)
    c = jnp.minimum(half_asinh, _ATMAX) * f / (nx + _EPS)
    a = c * w

    s0 = jnp.sum(a * y0, axis=1, keepdims=True)
    s1 = jnp.sum(a * y1, axis=1, keepdims=True)
    s = jnp.concatenate([s0, s1], axis=1) * th

    @pl.when(b == 0)
    def _init():
        carry_ref[...] = jnp.zeros_like(carry_ref)

    rK = jax.lax.broadcasted_iota(jnp.int32, (K, K), 0)
    cK = jax.lax.broadcasted_iota(jnp.int32, (K, K), 1)
    tril = (rK >= cK).astype(jnp.float32)
    S = jax.lax.dot_general(tril, s, (((1,), (0,)), ((), ())),
                            preferred_element_type=jnp.float32)
    S = S + carry_ref[0:1, 0:2]
    carry_ref[0:1, 0:2] = S[K - 1:K, :]

    SS = S * S
    nS = jnp.sqrt(SS[:, 0:1] + SS[:, 1:2])
    xd = jnp.tanh(nS) * S / (nS + _EPS)
    xx = xd * xd
    nxd2 = xx[:, 0:1] + xx[:, 1:2]
    out_ref[0] = 2.0 * xd / (1.0 - nxd2 + _EPS)


@jax.jit
def kernel(input, theta, class_w):
    B, N, C = input.shape
    K, Mdim = theta.shape
    inp_t = jnp.transpose(input, (0, 2, 1))
    out = pl.pallas_call(
        _pmanifold_body,
        grid=(B,),
        in_specs=[
            pl.BlockSpec(memory_space=pltpu.SMEM),
            pl.BlockSpec((1, C, N), lambda b: (b, 0, 0)),
            pl.BlockSpec((K, Mdim), lambda b: (0, 0)),
        ],
        out_specs=pl.BlockSpec((1, K, Mdim), lambda b: (b, 0, 0)),
        out_shape=jax.ShapeDtypeStruct((B, K, Mdim), jnp.float32),
        scratch_shapes=[pltpu.VMEM((8, 128), jnp.float32)],
        compiler_params=pltpu.CompilerParams(
            dimension_semantics=("arbitrary",)),
    )(class_w, inp_t, theta)
    return out.reshape(B, K * Mdim)


# speedup vs baseline: 2.6577x; 1.1218x over previous
import jax
import jax.numpy as jnp
from jax.experimental import pallas as pl
from jax.experimental.pallas import tpu as pltpu

_EPS = 1e-7
_ATMAX = 7.254340845282884


def _pmanifold_body(cw_ref, inp_ref, th_ref, out_ref, carry_ref):
    b = pl.program_id(0)
    _, _, N = inp_ref.shape
    K, Mdim = th_ref.shape

    ch = inp_ref[0]
    hom = ch[0:1, :]
    y0 = ch[1:2, :]
    y1 = ch[2:3, :]

    nz = (hom != 0.0) | (y0 != 0.0) | (y1 != 0.0)
    lane = jax.lax.broadcasted_iota(jnp.int32, (1, N), 1)
    first_zero = jnp.min(jnp.where(nz, N, lane))
    maskf = (lane < first_zero).astype(jnp.float32)

    w = jnp.where(hom > 0.5, cw_ref[1], cw_ref[0]) * maskf

    th = th_ref[...]
    thsq = th * th
    ysq = jnp.concatenate([y0 * y0, y1 * y1], axis=0)

    nu2 = jax.lax.dot_general(thsq, ysq, (((1,), (0,)), ((), ())),
                              preferred_element_type=jnp.float32)
    nu = jnp.sqrt(nu2)
    r = jnp.sqrt(1.0 + nu2)
    half_asinh = 0.5 * jnp.log(nu + r)
    c = jnp.minimum(half_asinh, _ATMAX) * (1.0 / (nu + (_EPS + _EPS * r)))
    a = c * w

    y01 = jnp.concatenate([y0, y1], axis=0)
    red = jax.lax.dot_general(a, y01, (((1,), (1,)), ((), ())),
                              preferred_element_type=jnp.float32)
    s = red * th

    @pl.when(b == 0)
    def _init():
        carry_ref[...] = jnp.zeros_like(carry_ref)

    rK = jax.lax.broadcasted_iota(jnp.int32, (K, K), 0)
    cK = jax.lax.broadcasted_iota(jnp.int32, (K, K), 1)
    tril = (rK >= cK).astype(jnp.float32)
    S = jax.lax.dot_general(tril, s, (((1,), (0,)), ((), ())),
                            preferred_element_type=jnp.float32)
    S = S + carry_ref[0:1, 0:2]
    carry_ref[0:1, 0:2] = S[K - 1:K, :]

    SS = S * S
    nS = jnp.sqrt(SS[:, 0:1] + SS[:, 1:2])
    xd = jnp.tanh(nS) * S / (nS + _EPS)
    xx = xd * xd
    nxd2 = xx[:, 0:1] + xx[:, 1:2]
    out_ref[0] = 2.0 * xd / (1.0 - nxd2 + _EPS)


@jax.jit
def kernel(input, theta, class_w):
    B, N, C = input.shape
    K, Mdim = theta.shape
    inp_t = jnp.transpose(input, (0, 2, 1))
    out = pl.pallas_call(
        _pmanifold_body,
        grid=(B,),
        in_specs=[
            pl.BlockSpec(memory_space=pltpu.SMEM),
            pl.BlockSpec((1, C, N), lambda b: (b, 0, 0)),
            pl.BlockSpec((K, Mdim), lambda b: (0, 0)),
        ],
        out_specs=pl.BlockSpec((1, K, Mdim), lambda b: (b, 0, 0)),
        out_shape=jax.ShapeDtypeStruct((B, K, Mdim), jnp.float32),
        scratch_shapes=[pltpu.VMEM((8, 128), jnp.float32)],
        compiler_params=pltpu.CompilerParams(
            dimension_semantics=("arbitrary",)),
    )(class_w, inp_t, theta)
    return out.reshape(B, K * Mdim)


# speedup vs baseline: 2.8128x; 1.0584x over previous
import jax
import jax.numpy as jnp
from jax.experimental import pallas as pl
from jax.experimental.pallas import tpu as pltpu

_EPS = 1e-7
_ATMAX2 = 14.508681690565768
_BC = 4


def _pmanifold_body(cw_ref, inp_ref, th_ref, out_ref, carry_ref):
    b = pl.program_id(0)
    _, _, N = inp_ref.shape
    K, Mdim = th_ref.shape

    ch = jnp.concatenate([inp_ref[i] for i in range(_BC)], axis=1)
    hom = ch[0:1, :]
    y0 = ch[1:2, :]
    y1 = ch[2:3, :]

    nz = (hom != 0.0) | (y0 != 0.0) | (y1 != 0.0)
    lane = jax.lax.broadcasted_iota(jnp.int32, (1, N), 1)
    wparts = []
    for i in range(_BC):
        nz_i = nz[:, i * N:(i + 1) * N]
        first_zero = jnp.min(jnp.where(nz_i, N, lane))
        wparts.append((lane < first_zero).astype(jnp.float32))
    maskf = jnp.concatenate(wparts, axis=1)

    w = jnp.where(hom > 0.5, cw_ref[1], cw_ref[0]) * maskf
    z0 = 0.5 * w * y0
    z1 = 0.5 * w * y1

    th = th_ref[...]
    thsq = th * th
    ysq = jnp.concatenate([y0 * y0, y1 * y1], axis=0)

    nu2 = jax.lax.dot_general(thsq, ysq, (((1,), (0,)), ((), ())),
                              preferred_element_type=jnp.float32)
    nu = jnp.sqrt(nu2)
    r = jnp.sqrt(1.0 + nu2)
    asinh_nu = jnp.log(nu + r)
    c = jnp.minimum(asinh_nu, _ATMAX2) * (1.0 / (nu + (_EPS + _EPS * r)))

    z01 = jnp.concatenate([z0, z1], axis=0)

    @pl.when(b == 0)
    def _init():
        carry_ref[...] = jnp.zeros_like(carry_ref)

    carry = carry_ref[0:1, 0:2]
    rK = jax.lax.broadcasted_iota(jnp.int32, (K, K), 0)
    cK = jax.lax.broadcasted_iota(jnp.int32, (K, K), 1)
    tril = (rK >= cK).astype(jnp.float32)

    for i in range(_BC):
        red = jax.lax.dot_general(
            c[:, i * N:(i + 1) * N], z01[:, i * N:(i + 1) * N],
            (((1,), (1,)), ((), ())), preferred_element_type=jnp.float32)
        s = red * th

        S = jax.lax.dot_general(tril, s, (((1,), (0,)), ((), ())),
                                preferred_element_type=jnp.float32)
        S = S + carry
        carry = S[K - 1:K, :]

        SS = S * S
        nS = jnp.sqrt(SS[:, 0:1] + SS[:, 1:2])
        xd = jnp.tanh(nS) * S / (nS + _EPS)
        xx = xd * xd
        nxd2 = xx[:, 0:1] + xx[:, 1:2]
        out_ref[i] = 2.0 * xd / (1.0 - nxd2 + _EPS)

    carry_ref[0:1, 0:2] = carry


@jax.jit
def kernel(input, theta, class_w):
    B, N, C = input.shape
    K, Mdim = theta.shape
    inp_t = jnp.transpose(input, (0, 2, 1))
    out = pl.pallas_call(
        _pmanifold_body,
        grid=(B // _BC,),
        in_specs=[
            pl.BlockSpec(memory_space=pltpu.SMEM),
            pl.BlockSpec((_BC, C, N), lambda b: (b, 0, 0)),
            pl.BlockSpec((K, Mdim), lambda b: (0, 0)),
        ],
        out_specs=pl.BlockSpec((_BC, K, Mdim), lambda b: (b, 0, 0)),
        out_shape=jax.ShapeDtypeStruct((B, K, Mdim), jnp.float32),
        scratch_shapes=[pltpu.VMEM((8, 128), jnp.float32)],
        compiler_params=pltpu.CompilerParams(
            dimension_semantics=("arbitrary",)),
    )(class_w, inp_t, theta)
    return out.reshape(B, K * Mdim)
